# baseline (speedup 1.0000x reference)
import functools
import math

import jax
import jax.numpy as jnp
from jax import lax
from jax.experimental import pallas as pl
from jax.experimental.pallas import tpu as pltpu
from jax.experimental.pallas import tpu_sc as plsc

N = 4194304
MAXS = 512
NBINS = MAXS + 1
BPAD = 521
LANES = 16
NC, NS = 2, 16
NW = NC * NS
PER_W = N // NW
CHUNK = 32768
NCHUNK = PER_W // CHUNK
VPC = CHUNK // LANES
HSIZE = LANES * BPAD
TROWS, TCOLS = 2048, 2048
TGRID = 16

_mesh = plsc.VectorSubcoreMesh(core_axis_name="c", subcore_axis_name="s")


@functools.partial(
    pl.kernel,
    out_type=jax.ShapeDtypeStruct((NW, HSIZE), jnp.float32),
    mesh=_mesh,
    scratch_types=(
        pltpu.VMEM((2, CHUNK), jnp.int32),
        pltpu.VMEM((HSIZE,), jnp.float32),
        pltpu.SemaphoreType.DMA,
    ),
    compiler_params=pltpu.CompilerParams(needs_layout_passes=False),
)
def _sc_hist(pred_hbm, hist_out, pred_v, hist_v, sem_p):
    wid = lax.axis_index("s") * NC + lax.axis_index("c")
    base = wid * PER_W

    lane_off = lax.iota(jnp.int32, LANES) * BPAD
    ones = jnp.full((LANES,), 1.0, jnp.float32)
    zeros = jnp.zeros((LANES,), jnp.float32)

    def _zero(j, carry):
        hist_v[pl.ds(j * LANES, LANES)] = zeros
        return carry
    lax.fori_loop(0, HSIZE // LANES, _zero, 0)

    def _chunk_body(buf):
        @plsc.parallel_loop(0, VPC, step=1, unroll=16)
        def _it(v):
            p = pred_v[buf, pl.ds(v * LANES, LANES)]
            plsc.addupdate_scatter(hist_v, [p + lane_off], ones)

    cps = [None, None]
    cps[0] = pltpu.async_copy(pred_hbm.at[pl.ds(base, CHUNK)], pred_v.at[0], sem_p)
    for c in range(NCHUNK):
        nb = (c + 1) % 2
        if c + 1 < NCHUNK:
            off = base + (c + 1) * CHUNK
            cps[nb] = pltpu.async_copy(pred_hbm.at[pl.ds(off, CHUNK)], pred_v.at[nb], sem_p)
        cb = c % 2
        cps[cb].wait()
        _chunk_body(cb)

    pltpu.sync_copy(hist_v, hist_out.at[wid])


def _mom_body(t_ref, out_ref):
    i = pl.program_id(0)

    @pl.when(i == 0)
    def _init():
        out_ref[0] = 0.0
        out_ref[1] = 0.0

    x = t_ref[...]
    out_ref[0] += jnp.sum(x)
    out_ref[1] += jnp.sum(x * x)


_tc_mom = pl.pallas_call(
    _mom_body,
    grid=(TGRID,),
    in_specs=[pl.BlockSpec((TROWS // TGRID, TCOLS), lambda i: (i, 0))],
    out_specs=pl.BlockSpec(memory_space=pltpu.SMEM),
    out_shape=jax.ShapeDtypeStruct((2,), jnp.float32),
)


def _loss_body(mom_ref, hp_ref, out_ref):
    n = jnp.float32(N)
    s = mom_ref[0]
    q = mom_ref[1]
    mu = s / n
    var = (q - s * mu) / (n - 1.0)
    sigma = jnp.sqrt(var)
    hist = jnp.sum(hp_ref[...], axis=0, keepdims=True)
    xi = lax.broadcasted_iota(jnp.int32, (1, BPAD), 1)
    xs = xi.astype(jnp.float32)
    mask = xi < NBINS
    z = (xs - mu) / sigma
    logp = -0.5 * z * z - jnp.log(sigma) - jnp.float32(0.5 * math.log(2.0 * math.pi))
    d = jnp.where(mask, jnp.exp(logp), 0.0)
    denom = jnp.maximum(jnp.sum(jnp.abs(hist)), 1e-12)
    pdist = hist / denom
    pw = jnp.where(mask, jnp.exp(d) * (d - pdist), 0.0)
    out_ref[...] = jnp.reshape(jnp.sum(pw) / jnp.float32(NBINS), (1, 1))


_tc_loss = pl.pallas_call(
    _loss_body,
    in_specs=[
        pl.BlockSpec(memory_space=pltpu.SMEM),
        pl.BlockSpec((NW * LANES, BPAD), lambda: (0, 0)),
    ],
    out_shape=jax.ShapeDtypeStruct((1, 1), jnp.float32),
)


def kernel(rt_pred, rt_true):
    hp = _sc_hist(rt_pred)
    mom = _tc_mom(rt_true.reshape(TROWS, TCOLS))
    out = _tc_loss(mom, hp.reshape(NW * LANES, BPAD))
    return out[0, 0]


# speedup vs baseline: 1.0083x; 1.0083x over previous
import functools
import math

import jax
import jax.numpy as jnp
from jax import lax
from jax.experimental import pallas as pl
from jax.experimental.pallas import tpu as pltpu
from jax.experimental.pallas import tpu_sc as plsc

N = 4194304
MAXS = 512
NBINS = MAXS + 1
BPAD = 521
LANES = 16
NC, NS = 2, 16
NW = NC * NS
PER_W = N // NW
CHUNK = 32768
NCHUNK = PER_W // CHUNK
VPC = CHUNK // LANES
HSIZE = LANES * BPAD
TROWS, TCOLS = 2048, 2048
TGRID = 16

_mesh = plsc.VectorSubcoreMesh(core_axis_name="c", subcore_axis_name="s")


@functools.partial(
    pl.kernel,
    out_type=jax.ShapeDtypeStruct((NW, HSIZE), jnp.float32),
    mesh=_mesh,
    scratch_types=(
        pltpu.VMEM((2, CHUNK), jnp.int32),
        pltpu.VMEM((HSIZE,), jnp.float32),
        pltpu.VMEM((HSIZE,), jnp.float32),
        pltpu.SemaphoreType.DMA,
    ),
    compiler_params=pltpu.CompilerParams(needs_layout_passes=False),
)
def _sc_hist(pred_hbm, hist_out, pred_v, hist_v, hist_w, sem_p):
    wid = lax.axis_index("s") * NC + lax.axis_index("c")
    base = wid * PER_W

    lane_off = lax.iota(jnp.int32, LANES) * BPAD
    ones = jnp.full((LANES,), 1.0, jnp.float32)
    zeros = jnp.zeros((LANES,), jnp.float32)

    def _zero(j, carry):
        hist_v[pl.ds(j * LANES, LANES)] = zeros
        hist_w[pl.ds(j * LANES, LANES)] = zeros
        return carry
    lax.fori_loop(0, HSIZE // LANES, _zero, 0)

    def _chunk_body(buf):
        @plsc.parallel_loop(0, VPC, step=2, unroll=4)
        def _it(v):
            p0 = pred_v[buf, pl.ds(v * LANES, LANES)]
            plsc.addupdate_scatter(hist_v, [p0 + lane_off], ones)
            p1 = pred_v[buf, pl.ds((v + 1) * LANES, LANES)]
            plsc.addupdate_scatter(hist_w, [p1 + lane_off], ones)

    cps = [None, None]
    cps[0] = pltpu.async_copy(pred_hbm.at[pl.ds(base, CHUNK)], pred_v.at[0], sem_p)
    for c in range(NCHUNK):
        nb = (c + 1) % 2
        if c + 1 < NCHUNK:
            off = base + (c + 1) * CHUNK
            cps[nb] = pltpu.async_copy(pred_hbm.at[pl.ds(off, CHUNK)], pred_v.at[nb], sem_p)
        cb = c % 2
        cps[cb].wait()
        _chunk_body(cb)

    @plsc.parallel_loop(0, HSIZE // LANES, step=1, unroll=8)
    def _fold(j):
        sl = pl.ds(j * LANES, LANES)
        hist_v[sl] = hist_v[sl] + hist_w[sl]

    pltpu.sync_copy(hist_v, hist_out.at[wid])


def _mom_body(t_ref, out_ref):
    i = pl.program_id(0)

    @pl.when(i == 0)
    def _init():
        out_ref[0] = 0.0
        out_ref[1] = 0.0

    x = t_ref[...]
    out_ref[0] += jnp.sum(x)
    out_ref[1] += jnp.sum(x * x)


_tc_mom = pl.pallas_call(
    _mom_body,
    grid=(TGRID,),
    in_specs=[pl.BlockSpec((TROWS // TGRID, TCOLS), lambda i: (i, 0))],
    out_specs=pl.BlockSpec(memory_space=pltpu.SMEM),
    out_shape=jax.ShapeDtypeStruct((2,), jnp.float32),
)


def _loss_body(mom_ref, hp_ref, out_ref):
    n = jnp.float32(N)
    s = mom_ref[0]
    q = mom_ref[1]
    mu = s / n
    var = (q - s * mu) / (n - 1.0)
    sigma = jnp.sqrt(var)
    hist = jnp.sum(hp_ref[...], axis=0, keepdims=True)
    xi = lax.broadcasted_iota(jnp.int32, (1, BPAD), 1)
    xs = xi.astype(jnp.float32)
    mask = xi < NBINS
    z = (xs - mu) / sigma
    logp = -0.5 * z * z - jnp.log(sigma) - jnp.float32(0.5 * math.log(2.0 * math.pi))
    d = jnp.where(mask, jnp.exp(logp), 0.0)
    denom = jnp.maximum(jnp.sum(jnp.abs(hist)), 1e-12)
    pdist = hist / denom
    pw = jnp.where(mask, jnp.exp(d) * (d - pdist), 0.0)
    out_ref[...] = jnp.reshape(jnp.sum(pw) / jnp.float32(NBINS), (1, 1))


_tc_loss = pl.pallas_call(
    _loss_body,
    in_specs=[
        pl.BlockSpec(memory_space=pltpu.SMEM),
        pl.BlockSpec((NW * LANES, BPAD), lambda: (0, 0)),
    ],
    out_shape=jax.ShapeDtypeStruct((1, 1), jnp.float32),
)


def kernel(rt_pred, rt_true):
    hp = _sc_hist(rt_pred)
    mom = _tc_mom(rt_true.reshape(TROWS, TCOLS))
    out = _tc_loss(mom, hp.reshape(NW * LANES, BPAD))
    return out[0, 0]


# speedup vs baseline: 1.2305x; 1.2204x over previous
import functools
import math

import jax
import jax.numpy as jnp
from jax import lax
from jax.experimental import pallas as pl
from jax.experimental.pallas import tpu as pltpu
from jax.experimental.pallas import tpu_sc as plsc

N = 4194304
MAXS = 512
NBINS = MAXS + 1
BPAD = 528
LANES = 16
NC, NS = 2, 16
NW = NC * NS
PER_W = N // NW
CHUNK = 32768
NCHUNK = PER_W // CHUNK
VPC = CHUNK // LANES
HSIZE = LANES * BPAD
TROWS, TCOLS = 32768, 128
TGRID = 16

_mesh = plsc.VectorSubcoreMesh(core_axis_name="c", subcore_axis_name="s")


@functools.partial(
    pl.kernel,
    out_type=jax.ShapeDtypeStruct((NW, BPAD), jnp.float32),
    mesh=_mesh,
    scratch_types=(
        pltpu.VMEM((2, CHUNK), jnp.int32),
        pltpu.VMEM((HSIZE,), jnp.float32),
        pltpu.VMEM((HSIZE,), jnp.float32),
        pltpu.VMEM((BPAD,), jnp.float32),
        pltpu.SemaphoreType.DMA,
    ),
    compiler_params=pltpu.CompilerParams(needs_layout_passes=False),
)
def _sc_hist(pred_hbm, hist_out, pred_v, hist_v, hist_w, fold_v, sem_p):
    wid = lax.axis_index("s") * NC + lax.axis_index("c")
    base = wid * PER_W

    lane_off = lax.iota(jnp.int32, LANES) * BPAD
    ones = jnp.full((LANES,), 1.0, jnp.float32)
    zeros = jnp.zeros((LANES,), jnp.float32)

    def _zero(j, carry):
        hist_v[pl.ds(j * LANES, LANES)] = zeros
        hist_w[pl.ds(j * LANES, LANES)] = zeros
        return carry
    lax.fori_loop(0, HSIZE // LANES, _zero, 0)

    def _chunk_body(buf):
        @plsc.parallel_loop(0, VPC, step=2, unroll=4)
        def _it(v):
            p0 = pred_v[buf, pl.ds(v * LANES, LANES)]
            plsc.addupdate_scatter(hist_v, [p0 + lane_off], ones)
            p1 = pred_v[buf, pl.ds((v + 1) * LANES, LANES)]
            plsc.addupdate_scatter(hist_w, [p1 + lane_off], ones)

    cps = [None, None]
    cps[0] = pltpu.async_copy(pred_hbm.at[pl.ds(base, CHUNK)], pred_v.at[0], sem_p)
    for c in range(NCHUNK):
        nb = (c + 1) % 2
        if c + 1 < NCHUNK:
            off = base + (c + 1) * CHUNK
            cps[nb] = pltpu.async_copy(pred_hbm.at[pl.ds(off, CHUNK)], pred_v.at[nb], sem_p)
        cb = c % 2
        cps[cb].wait()
        _chunk_body(cb)

    @plsc.parallel_loop(0, BPAD // LANES, step=1, unroll=2)
    def _fold(j):
        acc = jnp.zeros((LANES,), jnp.float32)
        for l in range(LANES):
            sl = pl.ds(l * BPAD + j * LANES, LANES)
            acc = acc + hist_v[sl] + hist_w[sl]
        fold_v[pl.ds(j * LANES, LANES)] = acc

    pltpu.sync_copy(fold_v, hist_out.at[wid])


def _mom_body(t_ref, out_ref):
    i = pl.program_id(0)

    @pl.when(i == 0)
    def _init():
        out_ref[0] = 0.0
        out_ref[1] = 0.0

    x = t_ref[...]
    out_ref[0] += jnp.sum(x)
    out_ref[1] += jnp.sum(x * x)


_tc_mom = pl.pallas_call(
    _mom_body,
    grid=(TGRID,),
    in_specs=[pl.BlockSpec((TROWS // TGRID, TCOLS), lambda i: (i, 0))],
    out_specs=pl.BlockSpec(memory_space=pltpu.SMEM),
    out_shape=jax.ShapeDtypeStruct((2,), jnp.float32),
)


def _loss_body(mom_ref, hp_ref, out_ref):
    n = jnp.float32(N)
    s = mom_ref[0]
    q = mom_ref[1]
    mu = s / n
    var = (q - s * mu) / (n - 1.0)
    sigma = jnp.sqrt(var)
    hist = jnp.sum(hp_ref[...], axis=0, keepdims=True)
    xi = lax.broadcasted_iota(jnp.int32, (1, BPAD), 1)
    xs = xi.astype(jnp.float32)
    mask = xi < NBINS
    z = (xs - mu) / sigma
    logp = -0.5 * z * z - jnp.log(sigma) - jnp.float32(0.5 * math.log(2.0 * math.pi))
    d = jnp.where(mask, jnp.exp(logp), 0.0)
    denom = jnp.maximum(jnp.sum(jnp.abs(hist)), 1e-12)
    pdist = hist / denom
    pw = jnp.where(mask, jnp.exp(d) * (d - pdist), 0.0)
    out_ref[...] = jnp.reshape(jnp.sum(pw) / jnp.float32(NBINS), (1, 1))


_tc_loss = pl.pallas_call(
    _loss_body,
    in_specs=[
        pl.BlockSpec(memory_space=pltpu.SMEM),
        pl.BlockSpec((NW, BPAD), lambda: (0, 0)),
    ],
    out_shape=jax.ShapeDtypeStruct((1, 1), jnp.float32),
)


def kernel(rt_pred, rt_true):
    hp = _sc_hist(rt_pred)
    mom = _tc_mom(rt_true.reshape(TROWS, TCOLS))
    out = _tc_loss(mom, hp)
    return out[0, 0]
